# baseline (speedup 1.0000x reference)
import jax
import jax.numpy as jnp
from jax import lax
from jax.experimental import pallas as pl
from jax.experimental.pallas import tpu as pltpu
from jax.experimental.pallas import tpu_sc as plsc

_N_NODES = 10000
_N_EDGES = 320000
_NC = 2
_NS = 16
_NW = _NC * _NS
_EPW = _N_EDGES // _NW
_CH = 80
_NCH = _EPW // _CH
_RPT = _N_NODES // _NS
_ZCH = 125
_NZ = _RPT // _ZCH

_EBLK = 3200
_EB8 = _EBLK // 8
_PBLK = 2000


def _lrelu(v):
    return jnp.where(v > 0, v, 0.01 * v)


def _node_body(x_ref, wn1, bn1, wn2, bn2, wg1n, out_ref):
    h = _lrelu(jnp.dot(x_ref[...], wn1[...],
                       preferred_element_type=jnp.float32) + bn1[...])
    ne = _lrelu(jnp.dot(h, wn2[...],
                        preferred_element_type=jnp.float32) + bn2[...])
    out_ref[...] = jnp.dot(ne, wg1n[...], preferred_element_type=jnp.float32)


def _node_proj(x, Wn1, bn1, Wn2, bn2, Wg1n):
    return pl.pallas_call(
        _node_body,
        out_shape=jax.ShapeDtypeStruct((_N_NODES, 16), jnp.float32),
    )(x, Wn1, bn1.reshape(1, -1), Wn2, bn2.reshape(1, -1), Wg1n)


def _gather_body(tbl_hbm, src_hbm, out_hbm, idx_v, buf_a, buf_b, sem_a,
                 sem_b):
    c = lax.axis_index("c")
    s = lax.axis_index("s")
    wid = c * _NS + s
    base = wid * _EPW
    pltpu.sync_copy(src_hbm.at[wid], idx_v)

    def start(i, buf, sem):
        return pltpu.async_copy(tbl_hbm.at[idx_v.at[i]], buf, sem)

    def drain(i, buf, sem):
        pltpu.make_async_copy(tbl_hbm.at[idx_v.at[i]], buf, sem).wait()
        pltpu.sync_copy(buf, out_hbm.at[pl.ds(base + i * _CH, _CH)])

    start(0, buf_a, sem_a)

    def pair(j, carry):
        start(2 * j + 1, buf_b, sem_b)
        drain(2 * j, buf_a, sem_a)
        start(2 * j + 2, buf_a, sem_a)
        drain(2 * j + 1, buf_b, sem_b)
        return carry

    lax.fori_loop(0, (_NCH - 1) // 2, pair, 0)
    drain(_NCH - 1, buf_a, sem_a)


def _gather(tbl, src_r):
    f = pl.kernel(
        _gather_body,
        out_type=jax.ShapeDtypeStruct((_N_EDGES, 16), jnp.float32),
        mesh=plsc.VectorSubcoreMesh(core_axis_name="c", subcore_axis_name="s"),
        compiler_params=pltpu.CompilerParams(use_tc_tiling_on_sc=False),
        scratch_types=[
            pltpu.VMEM((_NCH, _CH), jnp.int32),
            pltpu.VMEM((_CH, 16), jnp.float32),
            pltpu.VMEM((_CH, 16), jnp.float32),
            pltpu.SemaphoreType.DMA,
            pltpu.SemaphoreType.DMA,
        ],
    )
    return f(tbl, src_r)


def _msg_body(ea_ref, g_ref, w1, b1, w2, b2, w3, b3, w4, b4, out_ref):
    ea_p = jnp.concatenate(
        [ea_ref[a * _EB8:(a + 1) * _EB8, :] for a in range(8)], axis=1)
    e1 = _lrelu(jnp.dot(ea_p, w1[...],
                        preferred_element_type=jnp.float32) + b1[...])
    e2 = _lrelu(jnp.dot(e1, w2[...],
                        preferred_element_type=jnp.float32) + b2[...])
    h1 = _lrelu(jnp.dot(e2, w3[...],
                        preferred_element_type=jnp.float32)
                + g_ref[...] + b3[...])
    big = jnp.dot(h1, w4[...], preferred_element_type=jnp.float32)
    for s in range(8):
        out_ref[s * _EB8:(s + 1) * _EB8, :] = _lrelu(
            big[:, s * 128:(s + 1) * 128] + b4[...])


def _messages(ea, g_p, We1, be1, We2, be2, Wg1e, bg1, Wg2, bg2):
    eye8 = jnp.eye(8, dtype=jnp.float32)
    w1 = jnp.kron(eye8, We1)
    w2 = jnp.kron(eye8, We2)
    w3 = jnp.kron(eye8, Wg1e)
    w4 = jnp.kron(eye8, Wg2)
    b1 = jnp.tile(be1, 8).reshape(1, 128)
    b2 = jnp.tile(be2, 8).reshape(1, 512)
    b3 = jnp.tile(bg1, 8).reshape(1, 128)
    b4 = bg2.reshape(1, 128)
    nblk = _N_EDGES // _EBLK
    const = lambda shape: pl.BlockSpec(shape, lambda i: (0, 0))
    return pl.pallas_call(
        _msg_body,
        grid=(nblk,),
        in_specs=[
            pl.BlockSpec((_EBLK, 16), lambda i: (i, 0)),
            pl.BlockSpec((_EB8, 128), lambda i: (i, 0)),
            const((128, 128)), const((1, 128)),
            const((128, 512)), const((1, 512)),
            const((512, 128)), const((1, 128)),
            const((128, 1024)), const((1, 128)),
        ],
        out_specs=pl.BlockSpec((_EBLK, 128), lambda i: (i, 0)),
        out_shape=jax.ShapeDtypeStruct((_N_EDGES, 128), jnp.float32),
    )(ea, g_p, w1, b1, w2, b2, w3, b3, w4, b4)


def _scatter_body(msg_hbm, dst_hbm, out_hbm, idx_v, buf_a, buf_b, stage_v,
                  acc_sh, sem_a, sem_b):
    c = lax.axis_index("c")
    s = lax.axis_index("s")
    wid = c * _NS + s
    base = wid * _EPW

    def zrow(i, carry):
        for j in range(8):
            stage_v[i, pl.ds(j * 16, 16)] = jnp.zeros((16,), jnp.float32)
        return carry

    lax.fori_loop(0, _ZCH, zrow, 0)
    for k in range(_NZ):
        pltpu.sync_copy(stage_v, acc_sh.at[pl.ds(s * _RPT + k * _ZCH, _ZCH)])
    pltpu.sync_copy(dst_hbm.at[wid], idx_v)
    plsc.subcore_barrier()

    def start(i, buf, sem):
        return pltpu.async_copy(msg_hbm.at[pl.ds(base + i * _CH, _CH)], buf,
                                sem)

    def drain(i, buf, sem):
        pltpu.make_async_copy(msg_hbm.at[pl.ds(base + i * _CH, _CH)], buf,
                              sem).wait()
        pltpu.sync_copy(buf, acc_sh.at[idx_v.at[i]], add=True)

    start(0, buf_a, sem_a)

    def pair(j, carry):
        start(2 * j + 1, buf_b, sem_b)
        drain(2 * j, buf_a, sem_a)
        start(2 * j + 2, buf_a, sem_a)
        drain(2 * j + 1, buf_b, sem_b)
        return carry

    lax.fori_loop(0, (_NCH - 1) // 2, pair, 0)
    drain(_NCH - 1, buf_a, sem_a)
    plsc.subcore_barrier()

    for k in range(_NZ):
        row0 = s * _RPT + k * _ZCH
        pltpu.sync_copy(acc_sh.at[pl.ds(row0, _ZCH)], stage_v)
        pltpu.sync_copy(stage_v, out_hbm.at[c, pl.ds(row0, _ZCH)])


def _scatter(msg, dst_r):
    f = pl.kernel(
        _scatter_body,
        out_type=jax.ShapeDtypeStruct((_NC, _N_NODES, 128), jnp.float32),
        mesh=plsc.VectorSubcoreMesh(core_axis_name="c", subcore_axis_name="s"),
        compiler_params=pltpu.CompilerParams(use_tc_tiling_on_sc=False),
        scratch_types=[
            pltpu.VMEM((_NCH, _CH), jnp.int32),
            pltpu.VMEM((_CH, 128), jnp.float32),
            pltpu.VMEM((_CH, 128), jnp.float32),
            pltpu.VMEM((_ZCH, 128), jnp.float32),
            pltpu.VMEM_SHARED((_N_NODES, 128), jnp.float32),
            pltpu.SemaphoreType.DMA,
            pltpu.SemaphoreType.DMA,
        ],
    )
    return f(msg, dst_r)


def _post_body(parts_ref, x_ref, lng_g, lng_x, lnb_g, lnb_x,
               wp1g, wp1x, bp1, wp2, bp2, wp3, bp3, out_ref):
    gnn = parts_ref[0] + parts_ref[1]
    xv = x_ref[...]
    mu = (jnp.sum(gnn, axis=-1, keepdims=True)
          + jnp.sum(xv, axis=-1, keepdims=True)) * (1.0 / 256.0)
    cg = gnn - mu
    cx = xv - mu
    var = (jnp.sum(cg * cg, axis=-1, keepdims=True)
           + jnp.sum(cx * cx, axis=-1, keepdims=True)) * (1.0 / 256.0)
    rstd = lax.rsqrt(var + 1e-5)
    ng = cg * rstd * lng_g[...] + lnb_g[...]
    nx = cx * rstd * lng_x[...] + lnb_x[...]
    h = _lrelu(jnp.dot(ng, wp1g[...], preferred_element_type=jnp.float32)
               + jnp.dot(nx, wp1x[...], preferred_element_type=jnp.float32)
               + bp1[...])
    h = _lrelu(jnp.dot(h, wp2[...], preferred_element_type=jnp.float32)
               + bp2[...])
    out_ref[...] = (jnp.dot(h, wp3[...], preferred_element_type=jnp.float32)
                    + bp3[...])


def _post(parts, x, ln_g, ln_b, Wp1, bp1, Wp2, bp2, Wp3, bp3):
    nblk = _N_NODES // _PBLK
    const = lambda shape: pl.BlockSpec(shape, lambda i: tuple(0 for _ in shape))
    return pl.pallas_call(
        _post_body,
        grid=(nblk,),
        in_specs=[
            pl.BlockSpec((_NC, _PBLK, 128), lambda i: (0, i, 0)),
            pl.BlockSpec((_PBLK, 128), lambda i: (i, 0)),
            const((1, 128)), const((1, 128)), const((1, 128)), const((1, 128)),
            const((128, 32)), const((128, 32)), const((1, 32)),
            const((32, 32)), const((1, 32)),
            const((32, 128)), const((1, 128)),
        ],
        out_specs=pl.BlockSpec((_PBLK, 128), lambda i: (i, 0)),
        out_shape=jax.ShapeDtypeStruct((_N_NODES, 128), jnp.float32),
    )(parts, x,
      ln_g[:128].reshape(1, 128), ln_g[128:].reshape(1, 128),
      ln_b[:128].reshape(1, 128), ln_b[128:].reshape(1, 128),
      Wp1[:128], Wp1[128:], bp1.reshape(1, -1),
      Wp2, bp2.reshape(1, -1), Wp3, bp3.reshape(1, -1))


def kernel(x, edge_index, edge_attr, Wn1, bn1, Wn2, bn2, We1, be1, We2, be2,
           Wg1, bg1, Wg2, bg2, ln_g, ln_b, Wp1, bp1, Wp2, bp2, Wp3, bp3):
    src = (edge_index[0].astype(jnp.int32)
           .reshape(_N_EDGES // _EBLK, 8, _EB8)
           .transpose(0, 2, 1)
           .reshape(_NW, _NCH, _CH))
    dst = edge_index[1].astype(jnp.int32).reshape(_NW, _NCH, _CH)
    Wg1e = Wg1[:64]
    Wg1n = Wg1[64:]

    tbl = _node_proj(x, Wn1, bn1, Wn2, bn2, Wg1n)
    g_p = _gather(tbl, src).reshape(_N_EDGES // 8, 128)
    msg = _messages(edge_attr, g_p, We1, be1, We2, be2, Wg1e, bg1, Wg2, bg2)
    parts = _scatter(msg, dst)
    return _post(parts, x, ln_g, ln_b, Wp1, bp1, Wp2, bp2, Wp3, bp3)


# speedup vs baseline: 1.0843x; 1.0843x over previous
import functools

import jax
import jax.numpy as jnp
from jax import lax
from jax.experimental import pallas as pl
from jax.experimental.pallas import tpu as pltpu
from jax.experimental.pallas import tpu_sc as plsc

_N_NODES = 10000
_N_EDGES = 320000
_NC = 2
_NS = 16
_NW = _NC * _NS
_RPT = _N_NODES // _NS
_ZCH = 125
_NZ = _RPT // _ZCH

_EBLK = 3200
_EB8 = _EBLK // 8
_PBLK = 2000

_HALF_A = 166400
_HALF_B = 153600
_CFG = {
    _HALF_A: (80, 65),
    _HALF_B: (64, 75),
}


def _lrelu(v):
    return jnp.where(v > 0, v, 0.01 * v)


def _node_body(x_ref, wn1, bn1, wn2, bn2, wg1n, out_ref):
    h = _lrelu(jnp.dot(x_ref[...], wn1[...],
                       preferred_element_type=jnp.float32) + bn1[...])
    ne = _lrelu(jnp.dot(h, wn2[...],
                        preferred_element_type=jnp.float32) + bn2[...])
    out_ref[...] = jnp.dot(ne, wg1n[...], preferred_element_type=jnp.float32)


def _node_proj(x, Wn1, bn1, Wn2, bn2, Wg1n):
    return pl.pallas_call(
        _node_body,
        out_shape=jax.ShapeDtypeStruct((_N_NODES, 16), jnp.float32),
    )(x, Wn1, bn1.reshape(1, -1), Wn2, bn2.reshape(1, -1), Wg1n)


def _gather_body(n_e, ch, nch, tbl_hbm, src_hbm, out_hbm, idx_v, buf_a, buf_b,
                 sem_a, sem_b):
    c = lax.axis_index("c")
    s = lax.axis_index("s")
    wid = c * _NS + s
    epw = n_e // _NW
    base = wid * epw
    pltpu.sync_copy(src_hbm.at[wid], idx_v)

    def start(i, buf, sem):
        return pltpu.async_copy(tbl_hbm.at[idx_v.at[i]], buf, sem)

    def drain(i, buf, sem):
        pltpu.make_async_copy(tbl_hbm.at[idx_v.at[i]], buf, sem).wait()
        pltpu.sync_copy(buf, out_hbm.at[pl.ds(base + i * ch, ch)])

    start(0, buf_a, sem_a)

    def pair(j, carry):
        start(2 * j + 1, buf_b, sem_b)
        drain(2 * j, buf_a, sem_a)
        start(2 * j + 2, buf_a, sem_a)
        drain(2 * j + 1, buf_b, sem_b)
        return carry

    lax.fori_loop(0, (nch - 1) // 2, pair, 0)
    drain(nch - 1, buf_a, sem_a)


def _gather(tbl, src_r, n_e):
    ch, nch = _CFG[n_e]
    f = pl.kernel(
        functools.partial(_gather_body, n_e, ch, nch),
        out_type=jax.ShapeDtypeStruct((n_e, 16), jnp.float32),
        mesh=plsc.VectorSubcoreMesh(core_axis_name="c", subcore_axis_name="s"),
        compiler_params=pltpu.CompilerParams(use_tc_tiling_on_sc=False),
        scratch_types=[
            pltpu.VMEM((nch, ch), jnp.int32),
            pltpu.VMEM((ch, 16), jnp.float32),
            pltpu.VMEM((ch, 16), jnp.float32),
            pltpu.SemaphoreType.DMA,
            pltpu.SemaphoreType.DMA,
        ],
    )
    return f(tbl, src_r)


def _msg_body(ea_ref, g_ref, w1, b1, w2, b2, w3, b3, w4, b4, out_ref):
    e1 = _lrelu(jnp.dot(ea_ref[...], w1[...],
                        preferred_element_type=jnp.float32) + b1[...])
    e2 = _lrelu(jnp.dot(e1, w2[...],
                        preferred_element_type=jnp.float32) + b2[...])
    h1 = _lrelu(jnp.dot(e2, w3[...],
                        preferred_element_type=jnp.float32)
                + g_ref[...] + b3[...])
    big = jnp.dot(h1, w4[...], preferred_element_type=jnp.float32)
    for s in range(8):
        out_ref[s * _EB8:(s + 1) * _EB8, :] = _lrelu(
            big[:, s * 128:(s + 1) * 128] + b4[...])


def _messages(ea_p, blk0, g_p, n_e, We1, be1, We2, be2, Wg1e, bg1, Wg2, bg2):
    eye8 = jnp.eye(8, dtype=jnp.float32)
    w1 = jnp.kron(eye8, We1)
    w2 = jnp.kron(eye8, We2)
    w3 = jnp.kron(eye8, Wg1e)
    w4 = jnp.kron(eye8, Wg2)
    b1 = jnp.tile(be1, 8).reshape(1, 128)
    b2 = jnp.tile(be2, 8).reshape(1, 512)
    b3 = jnp.tile(bg1, 8).reshape(1, 128)
    b4 = bg2.reshape(1, 128)
    nblk = n_e // _EBLK
    const = lambda shape: pl.BlockSpec(shape, lambda i: (0, 0))
    return pl.pallas_call(
        _msg_body,
        grid=(nblk,),
        in_specs=[
            pl.BlockSpec((_EB8, 128), lambda i: (i + blk0, 0)),
            pl.BlockSpec((_EB8, 128), lambda i: (i, 0)),
            const((128, 128)), const((1, 128)),
            const((128, 512)), const((1, 512)),
            const((512, 128)), const((1, 128)),
            const((128, 1024)), const((1, 128)),
        ],
        out_specs=pl.BlockSpec((_EBLK, 128), lambda i: (i, 0)),
        out_shape=jax.ShapeDtypeStruct((n_e, 128), jnp.float32),
    )(ea_p, g_p, w1, b1, w2, b2, w3, b3, w4, b4)


def _scatter_body(n_e, ch, nch, msg_hbm, dst_hbm, out_hbm, idx_v, buf_a,
                  buf_b, stage_v, acc_sh, sem_a, sem_b):
    c = lax.axis_index("c")
    s = lax.axis_index("s")
    wid = c * _NS + s
    epw = n_e // _NW
    base = wid * epw

    def zrow(i, carry):
        for j in range(8):
            stage_v[i, pl.ds(j * 16, 16)] = jnp.zeros((16,), jnp.float32)
        return carry

    lax.fori_loop(0, _ZCH, zrow, 0)
    for k in range(_NZ):
        pltpu.sync_copy(stage_v, acc_sh.at[pl.ds(s * _RPT + k * _ZCH, _ZCH)])
    pltpu.sync_copy(dst_hbm.at[wid], idx_v)
    plsc.subcore_barrier()

    def start(i, buf, sem):
        return pltpu.async_copy(msg_hbm.at[pl.ds(base + i * ch, ch)], buf,
                                sem)

    def drain(i, buf, sem):
        pltpu.make_async_copy(msg_hbm.at[pl.ds(base + i * ch, ch)], buf,
                              sem).wait()
        pltpu.sync_copy(buf, acc_sh.at[idx_v.at[i]], add=True)

    start(0, buf_a, sem_a)

    def pair(j, carry):
        start(2 * j + 1, buf_b, sem_b)
        drain(2 * j, buf_a, sem_a)
        start(2 * j + 2, buf_a, sem_a)
        drain(2 * j + 1, buf_b, sem_b)
        return carry

    lax.fori_loop(0, (nch - 1) // 2, pair, 0)
    drain(nch - 1, buf_a, sem_a)
    plsc.subcore_barrier()

    for k in range(_NZ):
        row0 = s * _RPT + k * _ZCH
        pltpu.sync_copy(acc_sh.at[pl.ds(row0, _ZCH)], stage_v)
        pltpu.sync_copy(stage_v, out_hbm.at[c, pl.ds(row0, _ZCH)])


def _scatter(msg, dst_r, n_e):
    ch, nch = _CFG[n_e]
    f = pl.kernel(
        functools.partial(_scatter_body, n_e, ch, nch),
        out_type=jax.ShapeDtypeStruct((_NC, _N_NODES, 128), jnp.float32),
        mesh=plsc.VectorSubcoreMesh(core_axis_name="c", subcore_axis_name="s"),
        compiler_params=pltpu.CompilerParams(use_tc_tiling_on_sc=False),
        scratch_types=[
            pltpu.VMEM((nch, ch), jnp.int32),
            pltpu.VMEM((ch, 128), jnp.float32),
            pltpu.VMEM((ch, 128), jnp.float32),
            pltpu.VMEM((_ZCH, 128), jnp.float32),
            pltpu.VMEM_SHARED((_N_NODES, 128), jnp.float32),
            pltpu.SemaphoreType.DMA,
            pltpu.SemaphoreType.DMA,
        ],
    )
    return f(msg, dst_r)


def _post_body(pa_ref, pb_ref, x_ref, lng_g, lng_x, lnb_g, lnb_x,
               wp1g, wp1x, bp1, wp2, bp2, wp3, bp3, out_ref):
    gnn = (pa_ref[0] + pa_ref[1]) + (pb_ref[0] + pb_ref[1])
    xv = x_ref[...]
    mu = (jnp.sum(gnn, axis=-1, keepdims=True)
          + jnp.sum(xv, axis=-1, keepdims=True)) * (1.0 / 256.0)
    cg = gnn - mu
    cx = xv - mu
    var = (jnp.sum(cg * cg, axis=-1, keepdims=True)
           + jnp.sum(cx * cx, axis=-1, keepdims=True)) * (1.0 / 256.0)
    rstd = lax.rsqrt(var + 1e-5)
    ng = cg * rstd * lng_g[...] + lnb_g[...]
    nx = cx * rstd * lng_x[...] + lnb_x[...]
    h = _lrelu(jnp.dot(ng, wp1g[...], preferred_element_type=jnp.float32)
               + jnp.dot(nx, wp1x[...], preferred_element_type=jnp.float32)
               + bp1[...])
    h = _lrelu(jnp.dot(h, wp2[...], preferred_element_type=jnp.float32)
               + bp2[...])
    out_ref[...] = (jnp.dot(h, wp3[...], preferred_element_type=jnp.float32)
                    + bp3[...])


def _post(pa, pb, x, ln_g, ln_b, Wp1, bp1, Wp2, bp2, Wp3, bp3):
    nblk = _N_NODES // _PBLK
    const = lambda shape: pl.BlockSpec(shape, lambda i: tuple(0 for _ in shape))
    return pl.pallas_call(
        _post_body,
        grid=(nblk,),
        in_specs=[
            pl.BlockSpec((_NC, _PBLK, 128), lambda i: (0, i, 0)),
            pl.BlockSpec((_NC, _PBLK, 128), lambda i: (0, i, 0)),
            pl.BlockSpec((_PBLK, 128), lambda i: (i, 0)),
            const((1, 128)), const((1, 128)), const((1, 128)), const((1, 128)),
            const((128, 32)), const((128, 32)), const((1, 32)),
            const((32, 32)), const((1, 32)),
            const((32, 128)), const((1, 128)),
        ],
        out_specs=pl.BlockSpec((_PBLK, 128), lambda i: (i, 0)),
        out_shape=jax.ShapeDtypeStruct((_N_NODES, 128), jnp.float32),
    )(pa, pb, x,
      ln_g[:128].reshape(1, 128), ln_g[128:].reshape(1, 128),
      ln_b[:128].reshape(1, 128), ln_b[128:].reshape(1, 128),
      Wp1[:128], Wp1[128:], bp1.reshape(1, -1),
      Wp2, bp2.reshape(1, -1), Wp3, bp3.reshape(1, -1))


def _src_perm(src_half, n_e):
    ch, nch = _CFG[n_e]
    return (src_half.reshape(n_e // _EBLK, 8, _EB8)
            .transpose(0, 2, 1)
            .reshape(_NW, nch, ch))


def kernel(x, edge_index, edge_attr, Wn1, bn1, Wn2, bn2, We1, be1, We2, be2,
           Wg1, bg1, Wg2, bg2, ln_g, ln_b, Wp1, bp1, Wp2, bp2, Wp3, bp3):
    src = edge_index[0].astype(jnp.int32)
    dst = edge_index[1].astype(jnp.int32)
    src_a = _src_perm(src[:_HALF_A], _HALF_A)
    src_b = _src_perm(src[_HALF_A:], _HALF_B)
    dst_a = dst[:_HALF_A].reshape(_NW, *_CFG[_HALF_A][::-1])
    dst_b = dst[_HALF_A:].reshape(_NW, *_CFG[_HALF_B][::-1])
    Wg1e = Wg1[:64]
    Wg1n = Wg1[64:]

    tbl = _node_proj(x, Wn1, bn1, Wn2, bn2, Wg1n)
    g_a = _gather(tbl, src_a, _HALF_A).reshape(_HALF_A // 8, 128)
    g_b = _gather(tbl, src_b, _HALF_B).reshape(_HALF_B // 8, 128)
    ea_b, _ = lax.optimization_barrier((edge_attr, tbl))
    ea_p = (ea_b.reshape(_N_EDGES // _EBLK, 8, _EB8, 16)
            .transpose(0, 2, 1, 3)
            .reshape(_N_EDGES // 8, 128))
    enc = lambda gp, ne, b0: _messages(ea_p, b0, gp, ne, We1, be1, We2, be2,
                                       Wg1e, bg1, Wg2, bg2)
    msg_a = enc(g_a, _HALF_A, 0)
    pa = _scatter(msg_a, dst_a, _HALF_A)
    msg_b = enc(g_b, _HALF_B, _HALF_A // _EBLK)
    pb = _scatter(msg_b, dst_b, _HALF_B)
    return _post(pa, pb, x, ln_g, ln_b, Wp1, bp1, Wp2, bp2, Wp3, bp3)
